# baseline (speedup 1.0000x reference)
import functools

import jax
import jax.numpy as jnp
from jax import lax
from jax.experimental import pallas as pl
from jax.experimental.pallas import tpu as pltpu
from jax.experimental.pallas import tpu_sc as plsc

LANES = 16
NBUF = 2
NB = 2
BBLK = 128
EBLK = 8


def _make_sc_kernel(batch, maxlen, vocab, embed, n_workers):
    n_flat = batch * maxlen
    etiles = embed // EBLK
    btiles = batch // BBLK
    chunk = NB * BBLK
    blk_elems = EBLK * BBLK
    n_chunks = n_flat // (n_workers * chunk)
    chunks_per_m = btiles // NB
    kchunks = embed // LANES
    mesh = plsc.VectorSubcoreMesh(core_axis_name="c", subcore_axis_name="s")

    @functools.partial(
        pl.kernel,
        mesh=mesh,
        out_type=jax.ShapeDtypeStruct((maxlen * embed * batch,), jnp.float32),
        scratch_types=[
            [pltpu.VMEM((chunk,), jnp.int32) for _ in range(NBUF)],
            [pltpu.VMEM((chunk, embed), jnp.float32) for _ in range(NBUF)],
            [pltpu.VMEM((etiles * NB * blk_elems,), jnp.float32)
             for _ in range(NBUF)],
            pltpu.VMEM((maxlen, embed), jnp.float32),
            pltpu.VMEM((LANES + 2 * kchunks, LANES), jnp.int32),
            [pltpu.SemaphoreType.DMA for _ in range(NBUF)],
            [pltpu.SemaphoreType.DMA for _ in range(NBUF)],
        ],
        compiler_params=pltpu.CompilerParams(use_tc_tiling_on_sc=False,
                                             needs_layout_passes=False),
    )
    def sc_kernel(xt_hbm, tok_hbm, pos_hbm, out_hbm, idx_v, rows_v, p_v,
                  pos_v, tab_v, sem_g, sem_s):
        wid = lax.axis_index("s") * 2 + lax.axis_index("c")
        cbase = wid * n_chunks
        pltpu.sync_copy(pos_hbm, pos_v)

        iota = lax.iota(jnp.int32, LANES)
        for d in range(LANES):
            tab_v[d, :] = (d + iota) & (LANES - 1)
        for k in range(kchunks):
            tab_v[LANES + k, :] = (((2 * k + (iota >> 3)) << 11)
                                   + ((iota & 7) << 7))
            tab_v[LANES + kchunks + k, :] = k * LANES + iota

        def start_gather(b, c):
            m = c // chunks_per_m
            bt0 = (c % chunks_per_m) * NB
            for j in range(NB):
                src = ((m >> 3) * btiles + bt0 + j) * (8 * BBLK) \
                    + (m & 7) * BBLK
                pltpu.sync_copy(xt_hbm.at[pl.ds(src, BBLK)],
                                idx_v[b].at[pl.ds(j * BBLK, BBLK)])
            pltpu.async_copy(tok_hbm.at[idx_v[b]], rows_v[b], sem_g[b])

        def wait_gather(b):
            pltpu.make_async_copy(tok_hbm.at[idx_v[b]], rows_v[b],
                                  sem_g[b]).wait()

        def start_store(b, c):
            m = c // chunks_per_m
            bt0 = (c % chunks_per_m) * NB
            for et in range(etiles):
                dst = ((m * etiles + et) * btiles + bt0) * blk_elems
                pltpu.async_copy(
                    p_v[b].at[pl.ds(et * NB * blk_elems, NB * blk_elems)],
                    out_hbm.at[pl.ds(dst, NB * blk_elems)], sem_s[b])

        def wait_store(b, c):
            m = c // chunks_per_m
            bt0 = (c % chunks_per_m) * NB
            for et in range(etiles):
                dst = ((m * etiles + et) * btiles + bt0) * blk_elems
                pltpu.make_async_copy(
                    p_v[b].at[pl.ds(et * NB * blk_elems, NB * blk_elems)],
                    out_hbm.at[pl.ds(dst, NB * blk_elems)], sem_s[b]).wait()

        def transpose_add(b, c):
            m = c // chunks_per_m
            dvs = [tab_v[d, :] for d in range(LANES)]

            @plsc.parallel_loop(0, (BBLK // LANES) * NB * kchunks)
            def bb_body(i):
                bg = i >> 3
                j = (i >> 2) & (NB - 1)
                k = i & (kchunks - 1)
                row0 = j * BBLK + bg * LANES
                dst0 = j * blk_elems + bg * LANES
                pk = pos_v[m, pl.ds(k * LANES, LANES)]
                cik = tab_v[LANES + kchunks + k, :] + row0 * embed
                sbd = tab_v[LANES + k, :] + dst0
                for d in range(LANES):
                    val = plsc.load_gather(rows_v[b], [dvs[d], cik])
                    plsc.store_scatter(
                        p_v[b], [sbd + dvs[d]], val + pk)

        for b in range(NBUF):
            start_gather(b, cbase + b)

        n_outer = n_chunks // NBUF

        def outer(i, carry):
            c0 = cbase + i * NBUF
            for b in range(NBUF):
                c = c0 + b

                @pl.when(i > 0)
                def _():
                    wait_store(b, c - NBUF)

                wait_gather(b)
                transpose_add(b, c)
                start_store(b, c)

                @pl.when(i < n_outer - 1)
                def _():
                    start_gather(b, c + NBUF)

            return carry

        lax.fori_loop(0, n_outer, outer, 0)
        for b in range(NBUF):
            wait_store(b, cbase + n_chunks - NBUF + b)

    return sc_kernel


def kernel(x, token_table, pos_table):
    batch, maxlen = x.shape
    vocab, embed = token_table.shape
    n_workers = 32
    etiles = embed // EBLK
    btiles = batch // BBLK
    assert batch % BBLK == 0 and embed % EBLK == 0 and embed % LANES == 0
    assert (batch * maxlen) % (n_workers * NB * BBLK * NBUF) == 0
    assert (btiles // NB) * NB == btiles

    mt = maxlen // EBLK
    xt_flat = (x.T.astype(jnp.int32)
               .reshape(mt, EBLK, btiles, BBLK)
               .swapaxes(1, 2)
               .reshape(-1))
    sc = _make_sc_kernel(batch, maxlen, vocab, embed, n_workers)
    out = sc(xt_flat, token_table, pos_table)
    out = out.reshape(maxlen, etiles, btiles, EBLK, BBLK)
    out = out.transpose(2, 4, 0, 1, 3).reshape(batch, maxlen, embed)
    return out


# speedup vs baseline: 1.1447x; 1.1447x over previous
import functools

import jax
import jax.numpy as jnp
from jax import lax
from jax.experimental import pallas as pl
from jax.experimental.pallas import tpu as pltpu
from jax.experimental.pallas import tpu_sc as plsc

LANES = 16
NBUF = 2
NB = 2
BBLK = 128
EBLK = 8


def _make_sc_kernel(batch, maxlen, vocab, embed, n_workers):
    n_flat = batch * maxlen
    etiles = embed // EBLK
    btiles = batch // BBLK
    chunk = NB * BBLK
    blk_elems = EBLK * BBLK
    n_chunks = n_flat // (n_workers * chunk)
    chunks_per_m = btiles // NB
    kchunks = embed // LANES
    mesh = plsc.VectorSubcoreMesh(core_axis_name="c", subcore_axis_name="s")

    @functools.partial(
        pl.kernel,
        mesh=mesh,
        out_type=jax.ShapeDtypeStruct((maxlen * embed * batch,), jnp.float32),
        scratch_types=[
            [pltpu.VMEM((chunk,), jnp.int32) for _ in range(NBUF)],
            [pltpu.VMEM((chunk, embed), jnp.float32) for _ in range(NBUF)],
            [pltpu.VMEM((etiles * NB * blk_elems,), jnp.float32)
             for _ in range(NBUF)],
            pltpu.VMEM((maxlen, embed), jnp.float32),
            pltpu.VMEM((LANES + 2 * kchunks, LANES), jnp.int32),
            [pltpu.SemaphoreType.DMA for _ in range(NBUF)],
            [pltpu.SemaphoreType.DMA for _ in range(NBUF)],
        ],
        compiler_params=pltpu.CompilerParams(use_tc_tiling_on_sc=False,
                                             needs_layout_passes=False),
    )
    def sc_kernel(xt_hbm, tok_hbm, pos_hbm, out_hbm, idx_v, rows_v, p_v,
                  pos_v, tab_v, sem_g, sem_s):
        wid = lax.axis_index("s") * 2 + lax.axis_index("c")
        cbase = wid * n_chunks
        pltpu.sync_copy(pos_hbm, pos_v)

        iota = lax.iota(jnp.int32, LANES)
        for d in range(LANES):
            tab_v[d, :] = (d + iota) & (LANES - 1)
        for k in range(kchunks):
            tab_v[LANES + k, :] = (((2 * k + (iota >> 3)) << 11)
                                   + ((iota & 7) << 7))
            tab_v[LANES + kchunks + k, :] = k * LANES + iota

        def start_gather(b, c):
            m = c // chunks_per_m
            bt0 = (c % chunks_per_m) * NB
            for j in range(NB):
                src = ((m >> 3) * btiles + bt0 + j) * (8 * BBLK) \
                    + (m & 7) * BBLK
                pltpu.sync_copy(xt_hbm.at[pl.ds(src, BBLK)],
                                idx_v[b].at[pl.ds(j * BBLK, BBLK)])
            pltpu.async_copy(tok_hbm.at[idx_v[b]], rows_v[b], sem_g[b])

        def wait_gather(b):
            pltpu.make_async_copy(tok_hbm.at[idx_v[b]], rows_v[b],
                                  sem_g[b]).wait()

        def start_store(b, c):
            m = c // chunks_per_m
            bt0 = (c % chunks_per_m) * NB
            for et in range(etiles):
                dst = ((m * etiles + et) * btiles + bt0) * blk_elems
                pltpu.async_copy(
                    p_v[b].at[pl.ds(et * NB * blk_elems, NB * blk_elems)],
                    out_hbm.at[pl.ds(dst, NB * blk_elems)], sem_s[b])

        def wait_store(b, c):
            m = c // chunks_per_m
            bt0 = (c % chunks_per_m) * NB
            for et in range(etiles):
                dst = ((m * etiles + et) * btiles + bt0) * blk_elems
                pltpu.make_async_copy(
                    p_v[b].at[pl.ds(et * NB * blk_elems, NB * blk_elems)],
                    out_hbm.at[pl.ds(dst, NB * blk_elems)], sem_s[b]).wait()

        def transpose_add(b, c):
            m = c // chunks_per_m
            dvs = [tab_v[d, :] for d in range(LANES)]

            @plsc.parallel_loop(0, (BBLK // LANES) * NB * kchunks)
            def bb_body(i):
                bg = i >> 3
                j = (i >> 2) & (NB - 1)
                k = i & (kchunks - 1)
                row0 = j * BBLK + bg * LANES
                dst0 = j * blk_elems + bg * LANES
                pk = pos_v[m, pl.ds(k * LANES, LANES)]
                cik = tab_v[LANES + kchunks + k, :]
                sbd = tab_v[LANES + k, :] + dst0
                for d in range(LANES):
                    val = plsc.load_gather(
                        rows_v[b], [dvs[d] + row0, cik])
                    plsc.store_scatter(
                        p_v[b], [sbd + dvs[d]], val + pk)

        for b in range(NBUF):
            start_gather(b, cbase + b)

        n_outer = n_chunks // NBUF

        def outer(i, carry):
            c0 = cbase + i * NBUF
            for b in range(NBUF):
                c = c0 + b

                @pl.when(i > 0)
                def _():
                    wait_store(b, c - NBUF)

                wait_gather(b)
                transpose_add(b, c)
                start_store(b, c)

                @pl.when(i < n_outer - 1)
                def _():
                    start_gather(b, c + NBUF)

            return carry

        lax.fori_loop(0, n_outer, outer, 0)
        for b in range(NBUF):
            wait_store(b, cbase + n_chunks - NBUF + b)

    return sc_kernel


def kernel(x, token_table, pos_table):
    batch, maxlen = x.shape
    vocab, embed = token_table.shape
    n_workers = 32
    etiles = embed // EBLK
    btiles = batch // BBLK
    assert batch % BBLK == 0 and embed % EBLK == 0 and embed % LANES == 0
    assert (batch * maxlen) % (n_workers * NB * BBLK * NBUF) == 0
    assert (btiles // NB) * NB == btiles

    mt = maxlen // EBLK
    xt_flat = (x.T.astype(jnp.int32)
               .reshape(mt, EBLK, btiles, BBLK)
               .swapaxes(1, 2)
               .reshape(-1))
    sc = _make_sc_kernel(batch, maxlen, vocab, embed, n_workers)
    out = sc(xt_flat, token_table, pos_table)
    out = out.reshape(maxlen, etiles, btiles, EBLK, BBLK)
    out = out.transpose(2, 4, 0, 1, 3).reshape(batch, maxlen, embed)
    return out


# speedup vs baseline: 1.5881x; 1.3874x over previous
import functools

import jax
import jax.numpy as jnp
from jax import lax
from jax.experimental import pallas as pl
from jax.experimental.pallas import tpu as pltpu
from jax.experimental.pallas import tpu_sc as plsc

LANES = 16
NBUF = 2
NB = 2
BBLK = 128
EBLK = 8


def _make_sc_kernel(batch, maxlen, vocab, embed, n_workers):
    n_flat = batch * maxlen
    etiles = embed // EBLK
    btiles = batch // BBLK
    chunk = NB * BBLK
    blk_elems = EBLK * BBLK
    n_chunks = n_flat // (n_workers * chunk)
    chunks_per_m = btiles // NB
    kchunks = embed // LANES
    mesh = plsc.VectorSubcoreMesh(core_axis_name="c", subcore_axis_name="s")

    @functools.partial(
        pl.kernel,
        mesh=mesh,
        out_type=jax.ShapeDtypeStruct((maxlen * embed * batch,), jnp.float32),
        scratch_types=[
            [pltpu.VMEM((chunk,), jnp.int32) for _ in range(NBUF)],
            [pltpu.VMEM((chunk, embed), jnp.float32) for _ in range(NBUF)],
            [pltpu.VMEM((etiles * NB * blk_elems,), jnp.float32)
             for _ in range(NBUF)],
            pltpu.VMEM((maxlen, embed), jnp.float32),
            pltpu.VMEM((LANES + 2 * kchunks, LANES), jnp.int32),
            [pltpu.SemaphoreType.DMA for _ in range(NBUF)],
            [pltpu.SemaphoreType.DMA for _ in range(NBUF)],
            [pltpu.SemaphoreType.DMA for _ in range(NBUF)],
        ],
        compiler_params=pltpu.CompilerParams(use_tc_tiling_on_sc=False,
                                             needs_layout_passes=False),
    )
    def sc_kernel(xt_hbm, tok_hbm, pos_hbm, out_hbm, idx_v, rows_v, p_v,
                  pos_v, tab_v, sem_g, sem_s, sem_i):
        wid = lax.axis_index("s") * 2 + lax.axis_index("c")
        cbase = wid * n_chunks
        pltpu.sync_copy(pos_hbm, pos_v)

        iota = lax.iota(jnp.int32, LANES)
        for d in range(LANES):
            tab_v[d, :] = (d + iota) & (LANES - 1)
        for k in range(kchunks):
            tab_v[LANES + k, :] = (((2 * k + (iota >> 3)) << 11)
                                   + ((iota & 7) << 7))
            tab_v[LANES + kchunks + k, :] = k * LANES + iota

        def idx_copies(b, c):
            m = c // chunks_per_m
            bt0 = (c % chunks_per_m) * NB
            for j in range(NB):
                src = ((m >> 3) * btiles + bt0 + j) * (8 * BBLK) \
                    + (m & 7) * BBLK
                yield (xt_hbm.at[pl.ds(src, BBLK)],
                       idx_v[b].at[pl.ds(j * BBLK, BBLK)], sem_i[b])

        def start_idx(b, c):
            for args in idx_copies(b, c):
                pltpu.async_copy(*args)

        def wait_idx(b, c):
            for args in idx_copies(b, c):
                pltpu.make_async_copy(*args).wait()

        def start_gather(b):
            pltpu.async_copy(tok_hbm.at[idx_v[b]], rows_v[b], sem_g[b])

        def wait_gather(b):
            pltpu.make_async_copy(tok_hbm.at[idx_v[b]], rows_v[b],
                                  sem_g[b]).wait()

        def start_store(b, c):
            m = c // chunks_per_m
            bt0 = (c % chunks_per_m) * NB
            for et in range(etiles):
                dst = ((m * etiles + et) * btiles + bt0) * blk_elems
                pltpu.async_copy(
                    p_v[b].at[pl.ds(et * NB * blk_elems, NB * blk_elems)],
                    out_hbm.at[pl.ds(dst, NB * blk_elems)], sem_s[b])

        def wait_store(b, c):
            m = c // chunks_per_m
            bt0 = (c % chunks_per_m) * NB
            for et in range(etiles):
                dst = ((m * etiles + et) * btiles + bt0) * blk_elems
                pltpu.make_async_copy(
                    p_v[b].at[pl.ds(et * NB * blk_elems, NB * blk_elems)],
                    out_hbm.at[pl.ds(dst, NB * blk_elems)], sem_s[b]).wait()

        def transpose_add(b, c):
            m = c // chunks_per_m
            dvs = [tab_v[d, :] for d in range(LANES)]

            @plsc.parallel_loop(0, (BBLK // LANES) * NB * kchunks)
            def bb_body(i):
                bg = i >> 3
                j = (i >> 2) & (NB - 1)
                k = i & (kchunks - 1)
                row0 = j * BBLK + bg * LANES
                dst0 = j * blk_elems + bg * LANES
                pk = pos_v[m, pl.ds(k * LANES, LANES)]
                cik = tab_v[LANES + kchunks + k, :]
                sbd = tab_v[LANES + k, :] + dst0
                for d in range(LANES):
                    val = plsc.load_gather(
                        rows_v[b], [dvs[d] + row0, cik])
                    plsc.store_scatter(
                        p_v[b], [sbd + dvs[d]], val + pk)

        for b in range(NBUF):
            start_idx(b, cbase + b)
            wait_idx(b, cbase + b)
            start_gather(b)

        n_outer = n_chunks // NBUF

        def outer(i, carry):
            c0 = cbase + i * NBUF
            for b in range(NBUF):
                c = c0 + b

                @pl.when(i > 0)
                def _():
                    wait_store(b, c - NBUF)

                wait_gather(b)

                @pl.when(i < n_outer - 1)
                def _():
                    start_idx(b, c + NBUF)

                transpose_add(b, c)
                start_store(b, c)

                @pl.when(i < n_outer - 1)
                def _():
                    wait_idx(b, c + NBUF)
                    start_gather(b)

            return carry

        lax.fori_loop(0, n_outer, outer, 0)
        for b in range(NBUF):
            wait_store(b, cbase + n_chunks - NBUF + b)

    return sc_kernel


def kernel(x, token_table, pos_table):
    batch, maxlen = x.shape
    vocab, embed = token_table.shape
    n_workers = 32
    etiles = embed // EBLK
    btiles = batch // BBLK
    assert batch % BBLK == 0 and embed % EBLK == 0 and embed % LANES == 0
    assert (batch * maxlen) % (n_workers * NB * BBLK * NBUF) == 0
    assert (btiles // NB) * NB == btiles

    mt = maxlen // EBLK
    xt_flat = (x.T.astype(jnp.int32)
               .reshape(mt, EBLK, btiles, BBLK)
               .swapaxes(1, 2)
               .reshape(-1))
    sc = _make_sc_kernel(batch, maxlen, vocab, embed, n_workers)
    out = sc(xt_flat, token_table, pos_table)
    out = out.reshape(maxlen, etiles, btiles, EBLK, BBLK)
    out = out.transpose(2, 4, 0, 1, 3).reshape(batch, maxlen, embed)
    return out
